# baseline (speedup 1.0000x reference)
import jax
import jax.numpy as jnp
from jax.experimental import pallas as pl


def _identity_body(x_ref, o_ref):
    o_ref[...] = x_ref[...]


def _gcn(x, src, dst, n, W, b):
    h = x @ W
    deg = jnp.zeros((n,), x.dtype).at[dst].add(1.0)
    dinv = jnp.where(deg > 0, 1.0 / jnp.sqrt(deg), 0.0)
    norm = dinv[src] * dinv[dst]
    msg = h[src] * norm[:, None]
    out = jnp.zeros((n, W.shape[1]), x.dtype).at[dst].add(msg)
    return out + b


def kernel(noisy_coords, atom_types, noisy_edge_index, atom_emb, W1, b1, W2, b2, W3, b3, W4, b4, W5, b5):
    n = noisy_coords.shape[0]
    loop = jnp.arange(n, dtype=noisy_edge_index.dtype)
    src = jnp.concatenate([noisy_edge_index[0], loop])
    dst = jnp.concatenate([noisy_edge_index[1], loop])
    feats = jnp.concatenate([noisy_coords, atom_emb[atom_types]], axis=1)
    x = jax.nn.relu(_gcn(feats, src, dst, n, W1, b1))
    x = jax.nn.relu(_gcn(x, src, dst, n, W2, b2))
    x = jax.nn.relu(_gcn(x, src, dst, n, W3, b3))
    x = jax.nn.relu(_gcn(x, src, dst, n, W4, b4))
    x = _gcn(x, src, dst, n, W5, b5)
    return pl.pallas_call(
        _identity_body,
        out_shape=jax.ShapeDtypeStruct(x.shape, x.dtype),
    )(x)


# speedup vs baseline: 26.0834x; 26.0834x over previous
import functools

import jax
import jax.numpy as jnp
from jax import lax
from jax.experimental import pallas as pl
from jax.experimental.pallas import tpu as pltpu
from jax.experimental.pallas import tpu_sc as plsc

CHUNK = 128
BLK = 14
N_DUMP = 48
ZROWS = 1564
RB = 2000

_SC_PARAMS = pltpu.CompilerParams(use_tc_tiling_on_sc=False)


def _sc_pass(d, nblk, np_rows, with_gather):
    mesh = plsc.VectorSubcoreMesh(core_axis_name="c", subcore_axis_name="s")
    rpt = np_rows // 16

    scratch = [
        pltpu.VMEM((BLK, CHUNK), jnp.int32),
        pltpu.VMEM((BLK, CHUNK), jnp.int32),
        pltpu.VMEM((2, CHUNK, d), jnp.float32),
        pltpu.VMEM_SHARED((np_rows, d), jnp.float32),
        pltpu.SemaphoreType.DMA((2,)),
    ]

    @functools.partial(
        pl.kernel, mesh=mesh,
        out_type=jax.ShapeDtypeStruct((2, np_rows, d), jnp.float32),
        scratch_types=scratch, compiler_params=_SC_PARAMS)
    def scat(s_hbm, srcp_hbm, dstp_hbm, zeros_hbm, out_hbm,
             src_v, dst_v, rows_v, acc, gsem):
        c = lax.axis_index("c")
        s = lax.axis_index("s")
        tid = c * 16 + s

        for z in range(rpt // ZROWS):
            pltpu.sync_copy(zeros_hbm, acc.at[pl.ds(s * rpt + z * ZROWS,
                                                    ZROWS)])

        if with_gather:
            def outer(ob, carry):
                pltpu.sync_copy(srcp_hbm.at[tid, ob], src_v)
                pltpu.sync_copy(dstp_hbm.at[tid, ob], dst_v)
                pltpu.async_copy(s_hbm.at[src_v.at[0]], rows_v.at[0],
                                 gsem.at[0])
                for j in range(BLK):
                    b = j % 2
                    if j + 1 < BLK:
                        pltpu.async_copy(s_hbm.at[src_v.at[j + 1]],
                                         rows_v.at[1 - b], gsem.at[1 - b])
                    pltpu.make_async_copy(s_hbm.at[src_v.at[j]],
                                          rows_v.at[b], gsem.at[b]).wait()
                    pltpu.sync_copy(rows_v.at[b], acc.at[dst_v.at[j]],
                                    add=True)
                return carry
        else:
            def outer(ob, carry):
                pltpu.sync_copy(dstp_hbm.at[tid, ob], dst_v)
                for j in range(BLK):
                    pltpu.sync_copy(rows_v.at[0], acc.at[dst_v.at[j]],
                                    add=True)
                return carry

            pltpu.sync_copy(s_hbm, rows_v.at[0])

        plsc.subcore_barrier()
        lax.fori_loop(0, nblk, outer, 0)
        plsc.subcore_barrier()
        pltpu.sync_copy(acc.at[pl.ds(s * rpt, rpt)],
                        out_hbm.at[c, pl.ds(s * rpt, rpt)])

    return scat


def _row_spec(d):
    return pl.BlockSpec((RB, d), lambda i: (i, 0))


def _part_spec(d):
    return pl.BlockSpec((2, RB, d), lambda i: (0, i, 0))


def _full_spec(r, ncol):
    return pl.BlockSpec((r, ncol), lambda i: (0, 0))


def _tc_call(body, in_specs, out_specs, out_shape, grid):
    return pl.pallas_call(body, grid=grid, in_specs=in_specs,
                          out_specs=out_specs, out_shape=out_shape)


def _tc1_body(t_ref, coords_ref, atf_ref, e_ref, dinv_ref, sp1_ref):
    t = t_ref[...]
    deg = t[0, :, 0:1] + t[1, :, 0:1] + 1.0
    dinv = lax.rsqrt(deg)
    at = atf_ref[...]
    e = e_ref[...]
    emb = jnp.where(at < 0.5, e[0:1, 0:3], e[1:2, 0:3])
    feats = jnp.concatenate([coords_ref[...], emb], axis=1)
    sp = feats * dinv
    pad = jnp.zeros((sp.shape[0], 10), jnp.float32)
    sp1_ref[...] = jnp.concatenate([sp, pad], axis=1)
    dinv_ref[...] = dinv


def _layer_body(t_ref, sp_ref, dinv_ref, w_ref, b_ref, out_ref):
    t = t_ref[...]
    dinv = dinv_ref[...]
    u = dinv * (t[0] + t[1] + sp_ref[...])
    x = jnp.maximum(jnp.dot(u, w_ref[...],
                            preferred_element_type=jnp.float32) + b_ref[...],
                    0.0)
    out_ref[...] = x * dinv


def _tc3_body(t_ref, sp_ref, dinv_ref, w_ref, b_ref, oa_ref, ob_ref):
    t = t_ref[...]
    dinv = dinv_ref[...]
    u = dinv * (t[0] + t[1] + sp_ref[...])
    x = jnp.maximum(jnp.dot(u, w_ref[...],
                            preferred_element_type=jnp.float32) + b_ref[...],
                    0.0)
    sp = x * dinv
    oa_ref[...] = sp[:, 0:32]
    ob_ref[...] = sp[:, 32:64]


def _tc4_body(ta_ref, tb_ref, spa_ref, spb_ref, dinv_ref, w3_ref, b3_ref,
              w4_ref, out_ref):
    ta = ta_ref[...]
    tb = tb_ref[...]
    dinv = dinv_ref[...]
    ua = ta[0] + ta[1] + spa_ref[...]
    ub = tb[0] + tb[1] + spb_ref[...]
    u = dinv * jnp.concatenate([ua, ub], axis=1)
    x4 = jnp.maximum(jnp.dot(u, w3_ref[...],
                             preferred_element_type=jnp.float32) + b3_ref[...],
                     0.0)
    h4 = jnp.dot(x4, w4_ref[...], preferred_element_type=jnp.float32)
    out_ref[...] = h4 * dinv


def _tc5_body(t_ref, sp_ref, dinv_ref, b4_ref, w5_ref, out_ref):
    t = t_ref[...]
    dinv = dinv_ref[...]
    x5 = jnp.maximum(dinv * (t[0] + t[1] + sp_ref[...]) + b4_ref[...], 0.0)
    h5 = jnp.dot(x5, w5_ref[...], preferred_element_type=jnp.float32)
    out_ref[...] = h5 * dinv


def _tc6_body(t_ref, sp_ref, dinv_ref, b5_ref, out_ref):
    t = t_ref[...]
    dinv = dinv_ref[...]
    y = dinv * (t[0] + t[1] + sp_ref[...]) + b5_ref[...]
    out_ref[...] = y[:, 0:3]


def kernel(noisy_coords, atom_types, noisy_edge_index, atom_emb,
           W1, b1, W2, b2, W3, b3, W4, b4, W5, b5):
    n = noisy_coords.shape[0]
    e = noisy_edge_index.shape[1]
    np_rows = n + N_DUMP

    per_blk = BLK * CHUNK
    nblk = -(-e // (32 * per_blk))
    ep = 32 * nblk * per_blk
    padlen = ep - e
    src = noisy_edge_index[0]
    dst = noisy_edge_index[1]
    pad_i = jnp.arange(padlen, dtype=jnp.int32)
    src_p = jnp.concatenate([src, (pad_i * 9973) % n])
    dst_p = jnp.concatenate([dst, n + (pad_i % 8)])
    srcp = src_p.reshape(32, nblk, BLK, CHUNK)
    dstp = dst_p.reshape(32, nblk, BLK, CHUNK)

    zeros16 = jnp.zeros((ZROWS, 16), jnp.float32)
    zeros32 = jnp.zeros((ZROWS, 32), jnp.float32)
    ones16 = jnp.ones((CHUNK, 16), jnp.float32)

    deg_pass = _sc_pass(16, nblk, np_rows, with_gather=False)
    scat16 = _sc_pass(16, nblk, np_rows, with_gather=True)
    scat32 = _sc_pass(32, nblk, np_rows, with_gather=True)

    w1p = jnp.concatenate([W1, jnp.zeros((10, 32), jnp.float32)], axis=0)
    w5p = jnp.concatenate([W5, jnp.zeros((32, 13), jnp.float32)], axis=1)
    b1r = b1.reshape(1, -1)
    b2r = b2.reshape(1, -1)
    b3r = b3.reshape(1, -1)
    b4r = b4.reshape(1, -1)
    b5r = jnp.concatenate([b5, jnp.zeros((13,), jnp.float32)]).reshape(1, -1)
    e_pad = jnp.zeros((8, 128), jnp.float32).at[0:2, 0:3].set(atom_emb)
    atf = atom_types.astype(jnp.float32).reshape(n, 1)

    grid = (n // RB,)

    t_deg = deg_pass(ones16, srcp, dstp, zeros16)

    tc1 = _tc_call(
        _tc1_body,
        [_part_spec(16), _row_spec(3), _row_spec(1), _full_spec(8, 128)],
        [_row_spec(1), _row_spec(16)],
        (jax.ShapeDtypeStruct((n, 1), jnp.float32),
         jax.ShapeDtypeStruct((n, 16), jnp.float32)),
        grid)
    dinv, sp1 = tc1(t_deg, noisy_coords, atf, e_pad)

    t1 = scat16(sp1, srcp, dstp, zeros16)
    tc2 = _tc_call(
        _layer_body,
        [_part_spec(16), _row_spec(16), _row_spec(1), _full_spec(16, 32),
         _full_spec(1, 32)],
        _row_spec(32),
        jax.ShapeDtypeStruct((n, 32), jnp.float32),
        grid)
    sp2 = tc2(t1, sp1, dinv, w1p, b1r)

    t2 = scat32(sp2, srcp, dstp, zeros32)
    tc3 = _tc_call(
        _tc3_body,
        [_part_spec(32), _row_spec(32), _row_spec(1), _full_spec(32, 64),
         _full_spec(1, 64)],
        [_row_spec(32), _row_spec(32)],
        (jax.ShapeDtypeStruct((n, 32), jnp.float32),
         jax.ShapeDtypeStruct((n, 32), jnp.float32)),
        grid)
    sp3a, sp3b = tc3(t2, sp2, dinv, W2, b2r)

    t3a = scat32(sp3a, srcp, dstp, zeros32)
    t3b = scat32(sp3b, srcp, dstp, zeros32)
    tc4 = _tc_call(
        _tc4_body,
        [_part_spec(32), _part_spec(32), _row_spec(32), _row_spec(32),
         _row_spec(1), _full_spec(64, 64), _full_spec(1, 64),
         _full_spec(64, 32)],
        _row_spec(32),
        jax.ShapeDtypeStruct((n, 32), jnp.float32),
        grid)
    sp4 = tc4(t3a, t3b, sp3a, sp3b, dinv, W3, b3r, W4)

    t4 = scat32(sp4, srcp, dstp, zeros32)
    tc5 = _tc_call(
        _tc5_body,
        [_part_spec(32), _row_spec(32), _row_spec(1), _full_spec(1, 32),
         _full_spec(32, 16)],
        _row_spec(16),
        jax.ShapeDtypeStruct((n, 16), jnp.float32),
        grid)
    sp5 = tc5(t4, sp4, dinv, b4r, w5p)

    t5 = scat16(sp5, srcp, dstp, zeros16)
    tc6 = _tc_call(
        _tc6_body,
        [_part_spec(16), _row_spec(16), _row_spec(1), _full_spec(1, 16)],
        _row_spec(3),
        jax.ShapeDtypeStruct((n, 3), jnp.float32),
        grid)
    return tc6(t5, sp5, dinv, b5r)


# speedup vs baseline: 32.0104x; 1.2272x over previous
import functools

import jax
import jax.numpy as jnp
from jax import lax
from jax.experimental import pallas as pl
from jax.experimental.pallas import tpu as pltpu
from jax.experimental.pallas import tpu_sc as plsc

CHUNK = 128
NBUF = 4
IBLK = 28
N_DUMP = 48
ZROWS = 1564
RB = 2000

_SC_PARAMS = pltpu.CompilerParams(use_tc_tiling_on_sc=False)


def _sc_pass(d, nchunk, np_rows, with_gather):
    mesh = plsc.VectorSubcoreMesh(core_axis_name="c", subcore_axis_name="s")
    rpt = np_rows // 16
    nblk = nchunk // IBLK
    nsb = IBLK // NBUF

    scratch = [
        pltpu.VMEM((IBLK, CHUNK), jnp.int32),
        pltpu.VMEM((IBLK, CHUNK), jnp.int32),
        pltpu.VMEM((NBUF, CHUNK, d), jnp.float32),
        pltpu.VMEM_SHARED((np_rows, d), jnp.float32),
        pltpu.SemaphoreType.DMA((NBUF,)),
        pltpu.SemaphoreType.DMA((NBUF,)),
    ]

    @functools.partial(
        pl.kernel, mesh=mesh,
        out_type=jax.ShapeDtypeStruct((2, np_rows, d), jnp.float32),
        scratch_types=scratch, compiler_params=_SC_PARAMS)
    def scat(s_hbm, srcp_hbm, dstp_hbm, zeros_hbm, out_hbm,
             src_v, dst_v, rows_v, acc, gsem, ssem):
        c = lax.axis_index("c")
        s = lax.axis_index("s")
        tid = c * 16 + s

        for z in range(rpt // ZROWS):
            pltpu.sync_copy(zeros_hbm, acc.at[pl.ds(s * rpt + z * ZROWS,
                                                    ZROWS)])
        if not with_gather:
            pltpu.sync_copy(s_hbm, rows_v.at[0])
        plsc.subcore_barrier()

        if with_gather:
            def blk(ob, carry):
                pltpu.sync_copy(srcp_hbm.at[tid, ob], src_v)
                pltpu.sync_copy(dstp_hbm.at[tid, ob], dst_v)

                def sb(i, carry2):
                    base = i * NBUF
                    first = (ob == 0) & (i == 0)
                    for k in range(NBUF):
                        @pl.when(jnp.logical_not(first))
                        def _():
                            pltpu.make_async_copy(
                                rows_v.at[k], acc.at[dst_v.at[base + k]],
                                ssem.at[k]).wait()
                        pltpu.async_copy(s_hbm.at[src_v.at[base + k]],
                                         rows_v.at[k], gsem.at[k])
                    for k in range(NBUF):
                        pltpu.make_async_copy(s_hbm.at[src_v.at[base + k]],
                                              rows_v.at[k],
                                              gsem.at[k]).wait()
                        pltpu.async_copy(rows_v.at[k],
                                         acc.at[dst_v.at[base + k]],
                                         ssem.at[k], add=True)
                    return carry2

                return lax.fori_loop(0, nsb, sb, carry)

            lax.fori_loop(0, nblk, blk, 0)
            for k in range(NBUF):
                pltpu.make_async_copy(rows_v.at[k], acc.at[dst_v.at[k]],
                                      ssem.at[k]).wait()
        else:
            def blk(ob, carry):
                pltpu.sync_copy(dstp_hbm.at[tid, ob], dst_v)

                def sb(i, carry2):
                    base = i * NBUF
                    first = (ob == 0) & (i == 0)
                    for k in range(NBUF):
                        @pl.when(jnp.logical_not(first))
                        def _():
                            pltpu.make_async_copy(
                                rows_v.at[0], acc.at[dst_v.at[base + k]],
                                ssem.at[k]).wait()
                        pltpu.async_copy(rows_v.at[0],
                                         acc.at[dst_v.at[base + k]],
                                         ssem.at[k], add=True)
                    return carry2

                return lax.fori_loop(0, nsb, sb, carry)

            lax.fori_loop(0, nblk, blk, 0)
            for k in range(NBUF):
                pltpu.make_async_copy(rows_v.at[0], acc.at[dst_v.at[k]],
                                      ssem.at[k]).wait()

        plsc.subcore_barrier()
        pltpu.sync_copy(acc.at[pl.ds(s * rpt, rpt)],
                        out_hbm.at[c, pl.ds(s * rpt, rpt)])

    return scat


def _row_spec(d):
    return pl.BlockSpec((RB, d), lambda i: (i, 0))


def _part_spec(d):
    return pl.BlockSpec((2, RB, d), lambda i: (0, i, 0))


def _full_spec(r, ncol):
    return pl.BlockSpec((r, ncol), lambda i: (0, 0))


def _tc_call(body, in_specs, out_specs, out_shape, grid):
    return pl.pallas_call(body, grid=grid, in_specs=in_specs,
                          out_specs=out_specs, out_shape=out_shape)


def _tc1_body(t_ref, coords_ref, atf_ref, e_ref, dinv_ref, sp1_ref):
    t = t_ref[...]
    deg = t[0, :, 0:1] + t[1, :, 0:1] + 1.0
    dinv = lax.rsqrt(deg)
    at = atf_ref[...]
    e = e_ref[...]
    emb = jnp.where(at < 0.5, e[0:1, 0:3], e[1:2, 0:3])
    feats = jnp.concatenate([coords_ref[...], emb], axis=1)
    sp = feats * dinv
    pad = jnp.zeros((sp.shape[0], 10), jnp.float32)
    sp1_ref[...] = jnp.concatenate([sp, pad], axis=1)
    dinv_ref[...] = dinv


def _layer_body(t_ref, sp_ref, dinv_ref, w_ref, b_ref, out_ref):
    t = t_ref[...]
    dinv = dinv_ref[...]
    u = dinv * (t[0] + t[1] + sp_ref[...])
    x = jnp.maximum(jnp.dot(u, w_ref[...],
                            preferred_element_type=jnp.float32) + b_ref[...],
                    0.0)
    out_ref[...] = x * dinv


def _tc3_body(t_ref, sp_ref, dinv_ref, w_ref, b_ref, oa_ref, ob_ref):
    t = t_ref[...]
    dinv = dinv_ref[...]
    u = dinv * (t[0] + t[1] + sp_ref[...])
    x = jnp.maximum(jnp.dot(u, w_ref[...],
                            preferred_element_type=jnp.float32) + b_ref[...],
                    0.0)
    sp = x * dinv
    oa_ref[...] = sp[:, 0:32]
    ob_ref[...] = sp[:, 32:64]


def _tc4_body(ta_ref, tb_ref, spa_ref, spb_ref, dinv_ref, w3_ref, b3_ref,
              w4_ref, out_ref):
    ta = ta_ref[...]
    tb = tb_ref[...]
    dinv = dinv_ref[...]
    ua = ta[0] + ta[1] + spa_ref[...]
    ub = tb[0] + tb[1] + spb_ref[...]
    u = dinv * jnp.concatenate([ua, ub], axis=1)
    x4 = jnp.maximum(jnp.dot(u, w3_ref[...],
                             preferred_element_type=jnp.float32) + b3_ref[...],
                     0.0)
    h4 = jnp.dot(x4, w4_ref[...], preferred_element_type=jnp.float32)
    out_ref[...] = h4 * dinv


def _tc5_body(t_ref, sp_ref, dinv_ref, b4_ref, w5_ref, out_ref):
    t = t_ref[...]
    dinv = dinv_ref[...]
    x5 = jnp.maximum(dinv * (t[0] + t[1] + sp_ref[...]) + b4_ref[...], 0.0)
    h5 = jnp.dot(x5, w5_ref[...], preferred_element_type=jnp.float32)
    out_ref[...] = h5 * dinv


def _tc6_body(t_ref, sp_ref, dinv_ref, b5_ref, out_ref):
    t = t_ref[...]
    dinv = dinv_ref[...]
    y = dinv * (t[0] + t[1] + sp_ref[...]) + b5_ref[...]
    out_ref[...] = y[:, 0:3]


def kernel(noisy_coords, atom_types, noisy_edge_index, atom_emb,
           W1, b1, W2, b2, W3, b3, W4, b4, W5, b5):
    n = noisy_coords.shape[0]
    e = noisy_edge_index.shape[1]
    np_rows = n + N_DUMP

    nchunk = -(-e // (32 * CHUNK))
    nchunk += (-nchunk) % IBLK
    ep = 32 * nchunk * CHUNK
    padlen = ep - e
    src = noisy_edge_index[0]
    dst = noisy_edge_index[1]
    pad_i = jnp.arange(padlen, dtype=jnp.int32)
    src_p = jnp.concatenate([src, (pad_i * 9973) % n])
    dst_p = jnp.concatenate([dst, n + (pad_i % 8)])
    srcp = src_p.reshape(32, nchunk // IBLK, IBLK, CHUNK)
    dstp = dst_p.reshape(32, nchunk // IBLK, IBLK, CHUNK)

    zeros16 = jnp.zeros((ZROWS, 16), jnp.float32)
    zeros32 = jnp.zeros((ZROWS, 32), jnp.float32)
    ones16 = jnp.ones((CHUNK, 16), jnp.float32)

    deg_pass = _sc_pass(16, nchunk, np_rows, with_gather=False)
    scat16 = _sc_pass(16, nchunk, np_rows, with_gather=True)
    scat32 = _sc_pass(32, nchunk, np_rows, with_gather=True)

    w1p = jnp.concatenate([W1, jnp.zeros((10, 32), jnp.float32)], axis=0)
    w5p = jnp.concatenate([W5, jnp.zeros((32, 13), jnp.float32)], axis=1)
    b1r = b1.reshape(1, -1)
    b2r = b2.reshape(1, -1)
    b3r = b3.reshape(1, -1)
    b4r = b4.reshape(1, -1)
    b5r = jnp.concatenate([b5, jnp.zeros((13,), jnp.float32)]).reshape(1, -1)
    e_pad = jnp.zeros((8, 128), jnp.float32).at[0:2, 0:3].set(atom_emb)
    atf = atom_types.astype(jnp.float32).reshape(n, 1)

    grid = (n // RB,)

    t_deg = deg_pass(ones16, srcp, dstp, zeros16)

    tc1 = _tc_call(
        _tc1_body,
        [_part_spec(16), _row_spec(3), _row_spec(1), _full_spec(8, 128)],
        [_row_spec(1), _row_spec(16)],
        (jax.ShapeDtypeStruct((n, 1), jnp.float32),
         jax.ShapeDtypeStruct((n, 16), jnp.float32)),
        grid)
    dinv, sp1 = tc1(t_deg, noisy_coords, atf, e_pad)

    t1 = scat16(sp1, srcp, dstp, zeros16)
    tc2 = _tc_call(
        _layer_body,
        [_part_spec(16), _row_spec(16), _row_spec(1), _full_spec(16, 32),
         _full_spec(1, 32)],
        _row_spec(32),
        jax.ShapeDtypeStruct((n, 32), jnp.float32),
        grid)
    sp2 = tc2(t1, sp1, dinv, w1p, b1r)

    t2 = scat32(sp2, srcp, dstp, zeros32)
    tc3 = _tc_call(
        _tc3_body,
        [_part_spec(32), _row_spec(32), _row_spec(1), _full_spec(32, 64),
         _full_spec(1, 64)],
        [_row_spec(32), _row_spec(32)],
        (jax.ShapeDtypeStruct((n, 32), jnp.float32),
         jax.ShapeDtypeStruct((n, 32), jnp.float32)),
        grid)
    sp3a, sp3b = tc3(t2, sp2, dinv, W2, b2r)

    t3a = scat32(sp3a, srcp, dstp, zeros32)
    t3b = scat32(sp3b, srcp, dstp, zeros32)
    tc4 = _tc_call(
        _tc4_body,
        [_part_spec(32), _part_spec(32), _row_spec(32), _row_spec(32),
         _row_spec(1), _full_spec(64, 64), _full_spec(1, 64),
         _full_spec(64, 32)],
        _row_spec(32),
        jax.ShapeDtypeStruct((n, 32), jnp.float32),
        grid)
    sp4 = tc4(t3a, t3b, sp3a, sp3b, dinv, W3, b3r, W4)

    t4 = scat32(sp4, srcp, dstp, zeros32)
    tc5 = _tc_call(
        _tc5_body,
        [_part_spec(32), _row_spec(32), _row_spec(1), _full_spec(1, 32),
         _full_spec(32, 16)],
        _row_spec(16),
        jax.ShapeDtypeStruct((n, 16), jnp.float32),
        grid)
    sp5 = tc5(t4, sp4, dinv, b4r, w5p)

    t5 = scat16(sp5, srcp, dstp, zeros16)
    tc6 = _tc_call(
        _tc6_body,
        [_part_spec(16), _row_spec(16), _row_spec(1), _full_spec(1, 16)],
        _row_spec(3),
        jax.ShapeDtypeStruct((n, 3), jnp.float32),
        grid)
    return tc6(t5, sp5, dinv, b5r)
